# baseline (speedup 1.0000x reference)
import functools
import math

import jax
import jax.numpy as jnp
from jax.experimental import pallas as pl
from jax.experimental.pallas import tpu as pltpu

LABEL_SMOOTHING = 0.1
IGNORE_INDEX = -100
ROW_BLOCK = 128
COL_BLOCK = 6400


def _loss_body(x_ref, t_ref, o_ref, m_sc, se_sc, rs_sc, xt_sc, xz_sc,
               *, V, B, RB, CB):
    s = LABEL_SMOOTHING / (V - 2)
    c = 1.0 - LABEL_SMOOTHING
    Z = V + IGNORE_INDEX
    kc = (V - 2) * s * math.log(s) + c * math.log(c)
    s_log_s = s * math.log(s)
    nc = V // CB
    zc = Z // CB

    i = pl.program_id(0)
    jc = pl.program_id(1)
    t = t_ref[0]
    ch = x_ref[...]

    @pl.when(jc == 0)
    def _():
        m_sc[...] = jnp.full((RB, 1), -jnp.inf, dtype=jnp.float32)
        se_sc[...] = jnp.zeros((RB, 1), dtype=jnp.float32)
        rs_sc[...] = jnp.zeros((RB, 1), dtype=jnp.float32)
        xt_sc[...] = jnp.zeros((RB, 1), dtype=jnp.float32)

    m_old = m_sc[...]
    m_new = jnp.maximum(m_old, jnp.max(ch, axis=1, keepdims=True))
    se_sc[...] = (se_sc[...] * jnp.exp(m_old - m_new)
                  + jnp.sum(jnp.exp(ch - m_new), axis=1, keepdims=True))
    m_sc[...] = m_new
    rs_sc[...] += jnp.sum(ch, axis=1, keepdims=True)
    cols = jax.lax.broadcasted_iota(jnp.int32, (RB, CB), 1)
    xt_sc[...] += jnp.sum(jnp.where(cols == t - jc * CB, ch, 0.0),
                          axis=1, keepdims=True)

    @pl.when(jc == zc)
    def _():
        xz_sc[...] = ch[:, Z - zc * CB:Z - zc * CB + 1]

    @pl.when(jc == nc - 1)
    def _():
        a = m_sc[...] + jnp.log(se_sc[...])
        xz = xz_sc[...]
        lp_t = xt_sc[...] - a
        lp_z = xz - a
        ssum = rs_sc[...] - V * a
        loss = kc - s * ssum + (s - c) * lp_t + s * lp_z
        loss = loss + jnp.where(t == Z, s_log_s - s * lp_z, 0.0)
        loss = jnp.where(t == IGNORE_INDEX, 0.0, loss)
        part = jnp.sum(loss, keepdims=True) * (1.0 / B)

        @pl.when(i == 0)
        def _():
            o_ref[...] = jnp.zeros_like(o_ref)

        o_ref[...] += part


def kernel(output, target, one_hot):
    B, V = output.shape
    RB = ROW_BLOCK
    CB = COL_BLOCK
    G = B // RB
    C = V // CB
    t3 = target.reshape(G, RB, 1)
    out = pl.pallas_call(
        functools.partial(_loss_body, V=V, B=B, RB=RB, CB=CB),
        grid=(G, C),
        in_specs=[
            pl.BlockSpec((RB, CB), lambda i, j: (i, j)),
            pl.BlockSpec((1, RB, 1), lambda i, j: (i, 0, 0)),
        ],
        out_specs=pl.BlockSpec((1, 1), lambda i, j: (0, 0)),
        out_shape=jax.ShapeDtypeStruct((1, 1), jnp.float32),
        scratch_shapes=[
            pltpu.VMEM((RB, 1), jnp.float32),
            pltpu.VMEM((RB, 1), jnp.float32),
            pltpu.VMEM((RB, 1), jnp.float32),
            pltpu.VMEM((RB, 1), jnp.float32),
            pltpu.VMEM((RB, 1), jnp.float32),
        ],
    )(output, t3)
    return out[0, 0]


# speedup vs baseline: 1.6697x; 1.6697x over previous
import functools
import math

import jax
import jax.numpy as jnp
from jax.experimental import pallas as pl
from jax.experimental.pallas import tpu as pltpu

LABEL_SMOOTHING = 0.1
IGNORE_INDEX = -100
ROW_BLOCK = 128


def _loss_body(x_ref, t_ref, ts_ref, o_ref, xt_sc, *, V, B, RB):
    s = LABEL_SMOOTHING / (V - 2)
    c = 1.0 - LABEL_SMOOTHING
    Z = V + IGNORE_INDEX
    kc = (V - 2) * s * math.log(s) + c * math.log(c)
    s_log_s = s * math.log(s)

    i = pl.program_id(0)
    x = x_ref[...]
    t = t_ref[0]
    lane_ids = jax.lax.broadcasted_iota(jnp.int32, (1, 128), 1)
    for r in range(RB):
        tval = ts_ref[0, r, 0]
        gstart = pl.multiple_of((tval >> 7) << 7, 128)
        w = x_ref[pl.ds(r, 1), pl.ds(gstart, 128)]
        lane = tval & 127
        xt_sc[pl.ds(r, 1), :] = jnp.sum(
            jnp.where(lane_ids == lane, w, 0.0), axis=1, keepdims=True)
    xt = xt_sc[...]
    m = jnp.max(x, axis=1, keepdims=True)
    se = jnp.sum(jnp.exp(x - m), axis=1, keepdims=True)
    a = m + jnp.log(se)
    r_ = jnp.sum(x, axis=1, keepdims=True)
    xz = x[:, Z:Z + 1]
    lp_t = xt - a
    lp_z = xz - a
    ssum = r_ - V * a
    loss = kc - s * ssum + (s - c) * lp_t + s * lp_z
    loss = loss + jnp.where(t == Z, s_log_s - s * lp_z, 0.0)
    loss = jnp.where(t == IGNORE_INDEX, 0.0, loss)
    part = jnp.sum(loss, keepdims=True) * (1.0 / B)

    @pl.when(i == 0)
    def _():
        o_ref[...] = jnp.zeros_like(o_ref)

    o_ref[...] += part


def kernel(output, target, one_hot):
    B, V = output.shape
    RB = ROW_BLOCK
    G = B // RB
    t3 = target.reshape(G, RB, 1)
    out = pl.pallas_call(
        functools.partial(_loss_body, V=V, B=B, RB=RB),
        grid=(G,),
        in_specs=[
            pl.BlockSpec((RB, V), lambda i: (i, 0)),
            pl.BlockSpec((1, RB, 1), lambda i: (i, 0, 0)),
            pl.BlockSpec((1, RB, 1), lambda i: (i, 0, 0),
                         memory_space=pltpu.SMEM),
        ],
        out_specs=pl.BlockSpec((1, 1), lambda i: (0, 0)),
        out_shape=jax.ShapeDtypeStruct((1, 1), jnp.float32),
        scratch_shapes=[
            pltpu.VMEM((RB, 1), jnp.float32),
        ],
    )(output, t3, t3)
    return out[0, 0]
